# baseline (speedup 1.0000x reference)
import functools

import jax
import jax.numpy as jnp
from jax import lax
from jax.experimental import pallas as pl
from jax.experimental.pallas import tpu as pltpu
from jax.experimental.pallas import tpu_sc as plsc

V = 10000
NC, NS = 2, 16
NW = NC * NS


def _make_sc_kernel(B, S, L):
    PER_W = (B * S) // NW
    WPB = S // PER_W
    HIST = PER_W * V
    mesh = plsc.VectorSubcoreMesh(
        core_axis_name="c", subcore_axis_name="s",
        num_cores=NC, num_subcores=NS)

    @functools.partial(
        pl.kernel,
        out_type=jax.ShapeDtypeStruct((B * S, V), jnp.float32),
        mesh=mesh,
        compiler_params=pltpu.CompilerParams(
            needs_layout_passes=False, use_tc_tiling_on_sc=True),
        scratch_types=[
            pltpu.VMEM((L,), jnp.int32),
            pltpu.VMEM((L,), jnp.float32),
            pltpu.VMEM((2, S), jnp.int32),
            pltpu.VMEM((PER_W, V), jnp.float32),
            pltpu.VMEM_SHARED((V,), jnp.float32),
            pltpu.SemaphoreType.DMA,
        ],
    )
    def sc_kernel(tok_hbm, spans_hbm, tw_hbm, out_hbm,
                  tok_v, tw_v, spans_v, hist_v, zsp, zsem):
        c = lax.axis_index("c")
        s = lax.axis_index("s")
        w = s * NC + c
        b = w // WPB

        pltpu.sync_copy(tok_hbm.at[pl.ds(b * L, L)], tok_v)
        pltpu.sync_copy(tw_hbm.at[pl.ds(b * L, L)], tw_v)
        s0 = (w % WPB) * PER_W
        pltpu.sync_copy(spans_hbm.at[pl.ds(2 * b, 2)], spans_v)

        zeros = jnp.zeros((16,), jnp.float32)
        iota = lax.iota(jnp.int32, 16)
        lane_lt8 = iota < PER_W
        lane8 = jnp.where(lane_lt8, s0 + iota, 0)
        zero16 = jnp.zeros((16,), jnp.int32)
        iv = plsc.load_gather(spans_v, [zero16, lane8], mask=lane_lt8)
        jv = plsc.load_gather(spans_v, [zero16 + 1, lane8], mask=lane_lt8)

        @pl.when(s == 0)
        def _publish_zeros():
            @plsc.parallel_loop(0, V, 16, unroll=2)
            def z0(off):
                hist_v[0, pl.ds(pl.multiple_of(off, 16), 16)] = zeros

            pltpu.sync_copy(hist_v.at[0], zsp)

        plsc.subcore_barrier()
        for k in range(PER_W):
            pltpu.async_copy(zsp, hist_v.at[k], zsem)

        def span_body(k, carry):
            row = jnp.full((16,), k, jnp.int32)
            i = jnp.sum(jnp.where(iota == k, iv, 0))
            j = jnp.sum(jnp.where(iota == k, jv, 0))
            pltpu.make_async_copy(out_hbm.at[0], hist_v.at[k], zsem).wait()

            t0 = i // 16
            t1 = (j + 15) // 16

            @plsc.parallel_loop(t0 * 16, t1 * 16, 16, unroll=4)
            def sbody(off16):
                off = pl.multiple_of(off16, 16)
                pos = off + iota
                m = (pos >= i) & (pos < j)
                tok = tok_v[pl.ds(off, 16)]
                wv = tw_v[pl.ds(off, 16)]
                plsc.addupdate_scatter(hist_v, [row, tok], wv, mask=m)

            return carry

        lax.fori_loop(0, PER_W, span_body, None)
        pltpu.sync_copy(hist_v, out_hbm.at[pl.ds(w * PER_W, PER_W)])

    return sc_kernel


def kernel(tokens, lengths, span_idxs, token_weights):
    B = lengths.shape[0]
    L = tokens.shape[0] // B
    S = span_idxs.shape[1]
    spans_soa = span_idxs.transpose(0, 2, 1).reshape(2 * B, S)
    out = _make_sc_kernel(B, S, L)(tokens, spans_soa, token_weights)
    return out.reshape(B, S, V)


# speedup vs baseline: 1.0600x; 1.0600x over previous
import functools

import jax
import jax.numpy as jnp
from jax import lax
from jax.experimental import pallas as pl
from jax.experimental.pallas import tpu as pltpu
from jax.experimental.pallas import tpu_sc as plsc

V = 10000
NC, NS = 2, 16
NW = NC * NS


def _make_sc_kernel(B, S, L):
    PER_W = (B * S) // NW
    WPB = S // PER_W
    HIST = PER_W * V
    mesh = plsc.VectorSubcoreMesh(
        core_axis_name="c", subcore_axis_name="s",
        num_cores=NC, num_subcores=NS)

    @functools.partial(
        pl.kernel,
        out_type=jax.ShapeDtypeStruct((B * S, V), jnp.float32),
        mesh=mesh,
        compiler_params=pltpu.CompilerParams(
            needs_layout_passes=False, use_tc_tiling_on_sc=True),
        scratch_types=[
            pltpu.VMEM((L,), jnp.int32),
            pltpu.VMEM((L,), jnp.float32),
            pltpu.VMEM((2, S), jnp.int32),
            pltpu.VMEM((PER_W, V), jnp.float32),
        ],
    )
    def sc_kernel(tok_hbm, spans_hbm, tw_hbm, out_hbm,
                  tok_v, tw_v, spans_v, hist_v):
        c = lax.axis_index("c")
        s = lax.axis_index("s")
        w = s * NC + c
        b = w // WPB

        s0 = (w % WPB) * PER_W
        pltpu.sync_copy(tok_hbm.at[pl.ds(b * L, L)], tok_v)
        pltpu.sync_copy(tw_hbm.at[pl.ds(b * L, L)], tw_v)
        pltpu.sync_copy(spans_hbm.at[pl.ds(2 * b, 2)], spans_v)

        zeros = jnp.zeros((16,), jnp.float32)
        iota = lax.iota(jnp.int32, 16)
        lane_lt8 = iota < PER_W
        lane8 = jnp.where(lane_lt8, s0 + iota, 0)
        zero16 = jnp.zeros((16,), jnp.int32)
        iv = plsc.load_gather(spans_v, [zero16, lane8], mask=lane_lt8)
        jv = plsc.load_gather(spans_v, [zero16 + 1, lane8], mask=lane_lt8)

        @plsc.parallel_loop(0, V, 16, unroll=2)
        def zbody(off):
            o = pl.multiple_of(off, 16)
            for r in range(PER_W):
                hist_v[r, pl.ds(o, 16)] = zeros

        def span_body(k, carry):
            row = jnp.full((16,), k, jnp.int32)
            i = jnp.sum(jnp.where(iota == k, iv, 0))
            j = jnp.sum(jnp.where(iota == k, jv, 0))

            t0 = i // 16
            t1 = (j + 15) // 16

            @plsc.parallel_loop(t0 * 16, t1 * 16, 16, unroll=4)
            def sbody(off16):
                off = pl.multiple_of(off16, 16)
                pos = off + iota
                m = (pos >= i) & (pos < j)
                tok = tok_v[pl.ds(off, 16)]
                wv = tw_v[pl.ds(off, 16)]
                plsc.addupdate_scatter(hist_v, [row, tok], wv, mask=m)

            return carry

        lax.fori_loop(0, PER_W, span_body, None)
        pltpu.sync_copy(hist_v, out_hbm.at[pl.ds(w * PER_W, PER_W)])

    return sc_kernel


def kernel(tokens, lengths, span_idxs, token_weights):
    B = lengths.shape[0]
    L = tokens.shape[0] // B
    S = span_idxs.shape[1]
    spans_soa = span_idxs.transpose(0, 2, 1).reshape(2 * B, S)
    out = _make_sc_kernel(B, S, L)(tokens, spans_soa, token_weights)
    return out.reshape(B, S, V)
